# Initial kernel scaffold; baseline (speedup 1.0000x reference)
import jax
import jax.numpy as jnp
from jax.experimental import pallas as pl


def kernel(x, edge_index, batch, W1, b1, W2, b2):
    raise NotImplementedError("write your pallas kernel here")


# speedup vs baseline: 19.3352x; 19.3352x over previous
import functools

import jax
import jax.numpy as jnp
from jax import lax
from jax.experimental import pallas as pl
from jax.experimental.pallas import tpu as pltpu
from jax.experimental.pallas import tpu_sc as plsc

N_NODES = 10000
N_EDGES = 320000
D = 128
D2 = D // 2
NUM_GRAPHS = 128

NC = 2
NS = 16
K = 80
NCHUNK = 250
WB = 624
TAIL = N_NODES - NS * WB
TAIL_START = NS * WB
ZROWS = 208

_mesh = plsc.VectorSubcoreMesh(core_axis_name="c", subcore_axis_name="s")
_sc_params = pltpu.CompilerParams(use_tc_tiling_on_sc=False)


@functools.partial(
    pl.kernel,
    out_type=jax.ShapeDtypeStruct((NC, N_NODES, NS), jnp.float32),
    mesh=_mesh,
    scratch_types=[
        pltpu.VMEM((NCHUNK, K), jnp.int32),
        pltpu.VMEM((K, NS), jnp.float32),
        pltpu.VMEM((ZROWS, NS), jnp.float32),
        pltpu.VMEM_SHARED((N_NODES, NS), jnp.float32),
    ],
    compiler_params=_sc_params,
)
def _sc_degree(col_hbm, out_hbm, cv, ones_v, zb, hist):
    c = lax.axis_index("c")
    s = lax.axis_index("s")

    @pl.loop(0, K)
    def _(r):
        ones_v[r] = jnp.ones((NS,), jnp.float32)

    @pl.loop(0, ZROWS)
    def _(r):
        zb[r] = jnp.zeros((NS,), jnp.float32)

    @pl.loop(0, WB // ZROWS)
    def _(k):
        pltpu.sync_copy(zb, hist.at[pl.ds(s * WB + k * ZROWS, ZROWS)])

    @pl.when(s == 0)
    def _():
        pltpu.sync_copy(zb.at[pl.ds(0, TAIL)], hist.at[pl.ds(TAIL_START, TAIL)])

    pltpu.sync_copy(col_hbm.at[s], cv)
    plsc.subcore_barrier()

    @pl.loop(0, NCHUNK // NC)
    def _(j):
        pltpu.sync_copy(ones_v, hist.at[cv.at[c * (NCHUNK // NC) + j]], add=True)

    plsc.subcore_barrier()
    pltpu.sync_copy(hist.at[pl.ds(s * WB, WB)],
                    out_hbm.at[c, pl.ds(s * WB, WB)])

    @pl.when(s == 0)
    def _():
        pltpu.sync_copy(hist.at[pl.ds(TAIL_START, TAIL)],
                        out_hbm.at[c, pl.ds(TAIL_START, TAIL)])


@functools.partial(
    pl.kernel,
    out_type=jax.ShapeDtypeStruct((NC, N_NODES, D2), jnp.float32),
    mesh=_mesh,
    scratch_types=[
        pltpu.VMEM((NCHUNK, K), jnp.int32),
        pltpu.VMEM((NCHUNK, K), jnp.int32),
        pltpu.VMEM((K, D2), jnp.float32),
        pltpu.VMEM((K, D2), jnp.float32),
        pltpu.VMEM((ZROWS, D2), jnp.float32),
        pltpu.VMEM_SHARED((N_NODES, D2), jnp.float32),
        pltpu.SemaphoreType.DMA,
        pltpu.SemaphoreType.DMA,
    ],
    compiler_params=_sc_params,
)
def _sc_edge_pass(h_hbm, row_hbm, col_hbm, out_hbm,
                  rv, cv, buf_a, buf_b, zb, accum, sem_a, sem_b):
    c = lax.axis_index("c")
    s = lax.axis_index("s")
    hh = h_hbm.at[c]

    @pl.loop(0, ZROWS)
    def _(r):
        @pl.loop(0, D2, step=NS)
        def _(cc):
            zb[r, pl.ds(cc, NS)] = jnp.zeros((NS,), jnp.float32)

    @pl.loop(0, WB // ZROWS)
    def _(k):
        pltpu.sync_copy(zb, accum.at[pl.ds(s * WB + k * ZROWS, ZROWS)])

    @pl.when(s == 0)
    def _():
        pltpu.sync_copy(zb.at[pl.ds(0, TAIL)], accum.at[pl.ds(TAIL_START, TAIL)])

    pltpu.sync_copy(row_hbm.at[s], rv)
    pltpu.sync_copy(col_hbm.at[s], cv)
    plsc.subcore_barrier()

    @pl.loop(0, NCHUNK, step=2)
    def _(j):
        ca = pltpu.async_copy(hh.at[rv.at[j]], buf_a, sem_a)
        cb = pltpu.async_copy(hh.at[rv.at[j + 1]], buf_b, sem_b)
        ca.wait()
        pltpu.sync_copy(buf_a, accum.at[cv.at[j]], add=True)
        cb.wait()
        pltpu.sync_copy(buf_b, accum.at[cv.at[j + 1]], add=True)

    plsc.subcore_barrier()
    pltpu.sync_copy(accum.at[pl.ds(s * WB, WB)],
                    out_hbm.at[c, pl.ds(s * WB, WB)])

    @pl.when(s == 0)
    def _():
        pltpu.sync_copy(accum.at[pl.ds(TAIL_START, TAIL)],
                        out_hbm.at[c, pl.ds(TAIL_START, TAIL)])


_RB = 1000
_GRID = N_NODES // _RB


def _dis_block(hist_ref):
    deg = hist_ref[0, :, 0:1] + hist_ref[1, :, 0:1] + 1.0
    return lax.rsqrt(deg)


def _store_halves(out_ref, h):
    out_ref[0] = h[:, :D2]
    out_ref[1] = h[:, D2:]


def _cat_halves(p_ref, h_ref):
    return jnp.concatenate([p_ref[0] + h_ref[0], p_ref[1] + h_ref[1]], axis=1)


def _tc_h1_body(x_ref, w1t_ref, hist_ref, h1_ref):
    dis = _dis_block(hist_ref)
    h = jnp.dot(x_ref[...], w1t_ref[...],
                preferred_element_type=jnp.float32) * dis
    _store_halves(h1_ref, h)


def _tc_mid_body(p_ref, h1_ref, hist_ref, w2t_ref, b1_ref, h2_ref):
    dis = _dis_block(hist_ref)
    o1 = dis * _cat_halves(p_ref, h1_ref) + b1_ref[...]
    h2 = jnp.dot(o1, w2t_ref[...], preferred_element_type=jnp.float32) * dis
    _store_halves(h2_ref, h2)


def _tc_final_body(p_ref, h2_ref, hist_ref, b2_ref, batch_ref, pool_ref):
    dis = _dis_block(hist_ref)
    o2 = dis * _cat_halves(p_ref, h2_ref) + b2_ref[...]
    gids = lax.broadcasted_iota(jnp.int32, (_RB, NUM_GRAPHS), 1)
    p1hot = (batch_ref[...] == gids).astype(jnp.float32)

    @pl.when(pl.program_id(0) == 0)
    def _():
        pool_ref[...] = jnp.zeros_like(pool_ref)

    pool_ref[...] += lax.dot_general(
        p1hot, o2, (((0,), (0,)), ((), ())),
        preferred_element_type=jnp.float32)


_x_spec = pl.BlockSpec((_RB, D), lambda i: (i, 0))
_half_spec = pl.BlockSpec((NC, _RB, D2), lambda i: (0, i, 0))
_hist_spec = pl.BlockSpec((NC, _RB, NS), lambda i: (0, i, 0))
_w_spec = pl.BlockSpec((D, D), lambda i: (0, 0))
_b_spec = pl.BlockSpec((1, D), lambda i: (0, 0))
_half_shape = jax.ShapeDtypeStruct((NC, N_NODES, D2), jnp.float32)


def kernel(x, edge_index, batch, W1, b1, W2, b2):
    row = edge_index[0].astype(jnp.int32).reshape(NS, NCHUNK, K)
    col = edge_index[1].astype(jnp.int32).reshape(NS, NCHUNK, K)
    batch2d = batch.astype(jnp.int32).reshape(N_NODES, 1)
    w1t = W1.T
    w2t = W2.T
    b1r = b1.reshape(1, D)
    b2r = b2.reshape(1, D)

    hist = _sc_degree(col)

    h1 = pl.pallas_call(
        _tc_h1_body,
        grid=(_GRID,),
        in_specs=[_x_spec, _w_spec, _hist_spec],
        out_specs=_half_spec,
        out_shape=_half_shape,
    )(x, w1t, hist)

    s1 = _sc_edge_pass(h1, row, col)

    h2 = pl.pallas_call(
        _tc_mid_body,
        grid=(_GRID,),
        in_specs=[_half_spec, _half_spec, _hist_spec, _w_spec, _b_spec],
        out_specs=_half_spec,
        out_shape=_half_shape,
    )(s1, h1, hist, w2t, b1r)

    s2 = _sc_edge_pass(h2, row, col)

    pool = pl.pallas_call(
        _tc_final_body,
        grid=(_GRID,),
        in_specs=[_half_spec, _half_spec, _hist_spec, _b_spec,
                  pl.BlockSpec((_RB, 1), lambda i: (i, 0))],
        out_specs=pl.BlockSpec((NUM_GRAPHS, D), lambda i: (0, 0)),
        out_shape=jax.ShapeDtypeStruct((NUM_GRAPHS, D), jnp.float32),
    )(s2, h2, hist, b2r, batch2d)

    return pool


# speedup vs baseline: 31.5094x; 1.6296x over previous
import functools

import jax
import jax.numpy as jnp
from jax import lax
from jax.experimental import pallas as pl
from jax.experimental.pallas import tpu as pltpu
from jax.experimental.pallas import tpu_sc as plsc

N_NODES = 10000
N_EDGES = 320000
D = 128
D2 = D // 2
NUM_GRAPHS = 128

NC = 2
NS = 16
K = 80
NCHUNK = 250
WB = 624
TAIL = N_NODES - NS * WB
TAIL_START = NS * WB
ZROWS = 208

NBUF = 6
_NGRP = NCHUNK // NBUF
_NTAIL = NCHUNK - _NGRP * NBUF

_mesh = plsc.VectorSubcoreMesh(core_axis_name="c", subcore_axis_name="s")
_sc_params = pltpu.CompilerParams(use_tc_tiling_on_sc=False)


@functools.partial(
    pl.kernel,
    out_type=jax.ShapeDtypeStruct((NC, N_NODES, NS), jnp.float32),
    mesh=_mesh,
    scratch_types=[
        pltpu.VMEM((NCHUNK, K), jnp.int32),
        pltpu.VMEM((K, NS), jnp.float32),
        pltpu.VMEM((ZROWS, NS), jnp.float32),
        pltpu.VMEM_SHARED((N_NODES, NS), jnp.float32),
    ],
    compiler_params=_sc_params,
)
def _sc_degree(col_hbm, out_hbm, cv, ones_v, zb, hist):
    c = lax.axis_index("c")
    s = lax.axis_index("s")

    @pl.loop(0, K)
    def _(r):
        ones_v[r] = jnp.ones((NS,), jnp.float32)

    @pl.loop(0, ZROWS)
    def _(r):
        zb[r] = jnp.zeros((NS,), jnp.float32)

    @pl.loop(0, WB // ZROWS)
    def _(k):
        pltpu.sync_copy(zb, hist.at[pl.ds(s * WB + k * ZROWS, ZROWS)])

    @pl.when(s == 0)
    def _():
        pltpu.sync_copy(zb.at[pl.ds(0, TAIL)], hist.at[pl.ds(TAIL_START, TAIL)])

    pltpu.sync_copy(col_hbm.at[s], cv)
    plsc.subcore_barrier()

    @pl.loop(0, NCHUNK // NC)
    def _(j):
        pltpu.sync_copy(ones_v, hist.at[cv.at[c * (NCHUNK // NC) + j]], add=True)

    plsc.subcore_barrier()
    pltpu.sync_copy(hist.at[pl.ds(s * WB, WB)],
                    out_hbm.at[c, pl.ds(s * WB, WB)])

    @pl.when(s == 0)
    def _():
        pltpu.sync_copy(hist.at[pl.ds(TAIL_START, TAIL)],
                        out_hbm.at[c, pl.ds(TAIL_START, TAIL)])


@functools.partial(
    pl.kernel,
    out_type=jax.ShapeDtypeStruct((NC, N_NODES, D2), jnp.float32),
    mesh=_mesh,
    scratch_types=[
        pltpu.VMEM((NCHUNK, K), jnp.int32),
        pltpu.VMEM((NCHUNK, K), jnp.int32),
        pltpu.VMEM((NBUF, K, D2), jnp.float32),
        pltpu.VMEM((ZROWS, D2), jnp.float32),
        pltpu.VMEM_SHARED((N_NODES, D2), jnp.float32),
        pltpu.SemaphoreType.DMA((NBUF,)),
        pltpu.SemaphoreType.DMA((NBUF,)),
    ],
    compiler_params=_sc_params,
)
def _sc_edge_pass(h_hbm, row_hbm, col_hbm, out_hbm,
                  rv, cv, bufs, zb, accum, gsem, ssem):
    c = lax.axis_index("c")
    s = lax.axis_index("s")
    hh = h_hbm.at[c]

    @pl.loop(0, ZROWS)
    def _(r):
        @pl.loop(0, D2, step=NS)
        def _(cc):
            zb[r, pl.ds(cc, NS)] = jnp.zeros((NS,), jnp.float32)

    @pl.loop(0, WB // ZROWS)
    def _(k):
        pltpu.sync_copy(zb, accum.at[pl.ds(s * WB + k * ZROWS, ZROWS)])

    @pl.when(s == 0)
    def _():
        pltpu.sync_copy(zb.at[pl.ds(0, TAIL)], accum.at[pl.ds(TAIL_START, TAIL)])

    pltpu.sync_copy(row_hbm.at[s], rv)
    pltpu.sync_copy(col_hbm.at[s], cv)
    plsc.subcore_barrier()

    for b in range(NBUF):
        pltpu.async_copy(hh.at[rv.at[b]], bufs.at[b], gsem.at[b])

    @pl.loop(0, _NGRP)
    def _(g):
        for b in range(NBUF):
            j = g * NBUF + b
            pltpu.make_async_copy(hh.at[rv.at[j]], bufs.at[b], gsem.at[b]).wait()
            pltpu.async_copy(bufs.at[b], accum.at[cv.at[j]], ssem.at[b], add=True)
            nxt = j + NBUF

            @pl.when(nxt < NCHUNK)
            def _():
                pltpu.make_async_copy(
                    bufs.at[b], accum.at[cv.at[j]], ssem.at[b]).wait()
                pltpu.async_copy(hh.at[rv.at[nxt]], bufs.at[b], gsem.at[b])

    for t in range(_NTAIL):
        j = _NGRP * NBUF + t
        pltpu.make_async_copy(hh.at[rv.at[j]], bufs.at[t], gsem.at[t]).wait()
        pltpu.async_copy(bufs.at[t], accum.at[cv.at[j]], ssem.at[t], add=True)

    for b in range(NBUF):
        pltpu.make_async_copy(bufs.at[b], accum.at[cv.at[0]], ssem.at[b]).wait()

    plsc.subcore_barrier()
    pltpu.sync_copy(accum.at[pl.ds(s * WB, WB)],
                    out_hbm.at[c, pl.ds(s * WB, WB)])

    @pl.when(s == 0)
    def _():
        pltpu.sync_copy(accum.at[pl.ds(TAIL_START, TAIL)],
                        out_hbm.at[c, pl.ds(TAIL_START, TAIL)])


_RB = 1000
_GRID = N_NODES // _RB


def _dis_block(hist_ref):
    deg = hist_ref[0, :, 0:1] + hist_ref[1, :, 0:1] + 1.0
    return lax.rsqrt(deg)


def _store_halves(out_ref, h):
    out_ref[0] = h[:, :D2]
    out_ref[1] = h[:, D2:]


def _cat_halves(p_ref, h_ref):
    return jnp.concatenate([p_ref[0] + h_ref[0], p_ref[1] + h_ref[1]], axis=1)


def _tc_h1_body(x_ref, w1t_ref, hist_ref, h1_ref):
    dis = _dis_block(hist_ref)
    h = jnp.dot(x_ref[...], w1t_ref[...],
                preferred_element_type=jnp.float32) * dis
    _store_halves(h1_ref, h)


def _tc_mid_body(p_ref, h1_ref, hist_ref, w2t_ref, b1_ref, h2_ref):
    dis = _dis_block(hist_ref)
    o1 = dis * _cat_halves(p_ref, h1_ref) + b1_ref[...]
    h2 = jnp.dot(o1, w2t_ref[...], preferred_element_type=jnp.float32) * dis
    _store_halves(h2_ref, h2)


def _tc_final_body(p_ref, h2_ref, hist_ref, b2_ref, batch_ref, pool_ref):
    dis = _dis_block(hist_ref)
    o2 = dis * _cat_halves(p_ref, h2_ref) + b2_ref[...]
    gids = lax.broadcasted_iota(jnp.int32, (_RB, NUM_GRAPHS), 1)
    p1hot = (batch_ref[...] == gids).astype(jnp.float32)

    @pl.when(pl.program_id(0) == 0)
    def _():
        pool_ref[...] = jnp.zeros_like(pool_ref)

    pool_ref[...] += lax.dot_general(
        p1hot, o2, (((0,), (0,)), ((), ())),
        preferred_element_type=jnp.float32)


_x_spec = pl.BlockSpec((_RB, D), lambda i: (i, 0))
_half_spec = pl.BlockSpec((NC, _RB, D2), lambda i: (0, i, 0))
_hist_spec = pl.BlockSpec((NC, _RB, NS), lambda i: (0, i, 0))
_w_spec = pl.BlockSpec((D, D), lambda i: (0, 0))
_b_spec = pl.BlockSpec((1, D), lambda i: (0, 0))
_half_shape = jax.ShapeDtypeStruct((NC, N_NODES, D2), jnp.float32)


def kernel(x, edge_index, batch, W1, b1, W2, b2):
    row = edge_index[0].astype(jnp.int32).reshape(NS, NCHUNK, K)
    col = edge_index[1].astype(jnp.int32).reshape(NS, NCHUNK, K)
    batch2d = batch.astype(jnp.int32).reshape(N_NODES, 1)
    w1t = W1.T
    w2t = W2.T
    b1r = b1.reshape(1, D)
    b2r = b2.reshape(1, D)

    hist = _sc_degree(col)

    h1 = pl.pallas_call(
        _tc_h1_body,
        grid=(_GRID,),
        in_specs=[_x_spec, _w_spec, _hist_spec],
        out_specs=_half_spec,
        out_shape=_half_shape,
    )(x, w1t, hist)

    s1 = _sc_edge_pass(h1, row, col)

    h2 = pl.pallas_call(
        _tc_mid_body,
        grid=(_GRID,),
        in_specs=[_half_spec, _half_spec, _hist_spec, _w_spec, _b_spec],
        out_specs=_half_spec,
        out_shape=_half_shape,
    )(s1, h1, hist, w2t, b1r)

    s2 = _sc_edge_pass(h2, row, col)

    pool = pl.pallas_call(
        _tc_final_body,
        grid=(_GRID,),
        in_specs=[_half_spec, _half_spec, _hist_spec, _b_spec,
                  pl.BlockSpec((_RB, 1), lambda i: (i, 0))],
        out_specs=pl.BlockSpec((NUM_GRAPHS, D), lambda i: (0, 0)),
        out_shape=jax.ShapeDtypeStruct((NUM_GRAPHS, D), jnp.float32),
    )(s2, h2, hist, b2r, batch2d)

    return pool


# speedup vs baseline: 33.3359x; 1.0580x over previous
import functools

import jax
import jax.numpy as jnp
from jax import lax
from jax.experimental import pallas as pl
from jax.experimental.pallas import tpu as pltpu
from jax.experimental.pallas import tpu_sc as plsc

N_NODES = 10000
N_EDGES = 320000
D = 128
D2 = D // 2
NUM_GRAPHS = 128

NC = 2
NS = 16
K = 80
NCHUNK = 250
WB = 624
TAIL = N_NODES - NS * WB
TAIL_START = NS * WB
ZROWS = 208

NBUF = 6
_NGRP = NCHUNK // NBUF
_NTAIL = NCHUNK - _NGRP * NBUF

_mesh = plsc.VectorSubcoreMesh(core_axis_name="c", subcore_axis_name="s")
_sc_params = pltpu.CompilerParams(use_tc_tiling_on_sc=False)


@functools.partial(
    pl.kernel,
    out_type=jax.ShapeDtypeStruct((NC, N_NODES, NS), jnp.float32),
    mesh=_mesh,
    scratch_types=[
        pltpu.VMEM((NCHUNK, K), jnp.int32),
        pltpu.VMEM((K, NS), jnp.float32),
        pltpu.VMEM((ZROWS, NS), jnp.float32),
        pltpu.VMEM_SHARED((N_NODES, NS), jnp.float32),
    ],
    compiler_params=_sc_params,
)
def _sc_degree(edge_hbm, out_hbm, cv, ones_v, zb, hist):
    c = lax.axis_index("c")
    s = lax.axis_index("s")

    @pl.loop(0, K)
    def _(r):
        ones_v[r] = jnp.ones((NS,), jnp.float32)

    @pl.loop(0, ZROWS)
    def _(r):
        zb[r] = jnp.zeros((NS,), jnp.float32)

    @pl.loop(0, WB // ZROWS)
    def _(k):
        pltpu.sync_copy(zb, hist.at[pl.ds(s * WB + k * ZROWS, ZROWS)])

    @pl.when(s == 0)
    def _():
        pltpu.sync_copy(zb.at[pl.ds(0, TAIL)], hist.at[pl.ds(TAIL_START, TAIL)])

    pltpu.sync_copy(edge_hbm.at[1, s], cv)
    plsc.subcore_barrier()

    @pl.loop(0, NCHUNK // NC)
    def _(j):
        pltpu.sync_copy(ones_v, hist.at[cv.at[c * (NCHUNK // NC) + j]], add=True)

    plsc.subcore_barrier()
    pltpu.sync_copy(hist.at[pl.ds(s * WB, WB)],
                    out_hbm.at[c, pl.ds(s * WB, WB)])

    @pl.when(s == 0)
    def _():
        pltpu.sync_copy(hist.at[pl.ds(TAIL_START, TAIL)],
                        out_hbm.at[c, pl.ds(TAIL_START, TAIL)])


@functools.partial(
    pl.kernel,
    out_type=jax.ShapeDtypeStruct((NC, N_NODES, D2), jnp.float32),
    mesh=_mesh,
    scratch_types=[
        pltpu.VMEM((NCHUNK, K), jnp.int32),
        pltpu.VMEM((NCHUNK, K), jnp.int32),
        pltpu.VMEM((NBUF, K, D2), jnp.float32),
        pltpu.VMEM((ZROWS, D2), jnp.float32),
        pltpu.VMEM_SHARED((N_NODES, D2), jnp.float32),
        pltpu.SemaphoreType.DMA((NBUF,)),
        pltpu.SemaphoreType.DMA((NBUF,)),
    ],
    compiler_params=_sc_params,
)
def _sc_edge_pass(h_hbm, edge_hbm, out_hbm,
                  rv, cv, bufs, zb, accum, gsem, ssem):
    c = lax.axis_index("c")
    s = lax.axis_index("s")
    hh = h_hbm.at[c]

    @pl.loop(0, ZROWS)
    def _(r):
        @pl.loop(0, D2, step=NS)
        def _(cc):
            zb[r, pl.ds(cc, NS)] = jnp.zeros((NS,), jnp.float32)

    @pl.loop(0, WB // ZROWS)
    def _(k):
        pltpu.sync_copy(zb, accum.at[pl.ds(s * WB + k * ZROWS, ZROWS)])

    @pl.when(s == 0)
    def _():
        pltpu.sync_copy(zb.at[pl.ds(0, TAIL)], accum.at[pl.ds(TAIL_START, TAIL)])

    pltpu.sync_copy(edge_hbm.at[0, s], rv)
    pltpu.sync_copy(edge_hbm.at[1, s], cv)
    plsc.subcore_barrier()

    for b in range(NBUF):
        pltpu.async_copy(hh.at[rv.at[b]], bufs.at[b], gsem.at[b])

    @pl.loop(0, _NGRP)
    def _(g):
        for b in range(NBUF):
            j = g * NBUF + b
            pltpu.make_async_copy(hh.at[rv.at[j]], bufs.at[b], gsem.at[b]).wait()
            pltpu.async_copy(bufs.at[b], accum.at[cv.at[j]], ssem.at[b], add=True)
            nxt = j + NBUF

            @pl.when(nxt < NCHUNK)
            def _():
                pltpu.make_async_copy(
                    bufs.at[b], accum.at[cv.at[j]], ssem.at[b]).wait()
                pltpu.async_copy(hh.at[rv.at[nxt]], bufs.at[b], gsem.at[b])

    for t in range(_NTAIL):
        j = _NGRP * NBUF + t
        pltpu.make_async_copy(hh.at[rv.at[j]], bufs.at[t], gsem.at[t]).wait()
        pltpu.async_copy(bufs.at[t], accum.at[cv.at[j]], ssem.at[t], add=True)

    for b in range(NBUF):
        pltpu.make_async_copy(bufs.at[b], accum.at[cv.at[0]], ssem.at[b]).wait()

    plsc.subcore_barrier()
    pltpu.sync_copy(accum.at[pl.ds(s * WB, WB)],
                    out_hbm.at[c, pl.ds(s * WB, WB)])

    @pl.when(s == 0)
    def _():
        pltpu.sync_copy(accum.at[pl.ds(TAIL_START, TAIL)],
                        out_hbm.at[c, pl.ds(TAIL_START, TAIL)])


_RB = 2000
_GRID = N_NODES // _RB


def _dis_block(hist_ref):
    deg = hist_ref[0, :, 0:1] + hist_ref[1, :, 0:1] + 1.0
    return lax.rsqrt(deg)


def _store_halves(out_ref, h):
    out_ref[0] = h[:, :D2]
    out_ref[1] = h[:, D2:]


def _cat_halves(p_ref, h_ref):
    return jnp.concatenate([p_ref[0] + h_ref[0], p_ref[1] + h_ref[1]], axis=1)


def _tc_h1_body(x_ref, w1t_ref, hist_ref, h1_ref):
    dis = _dis_block(hist_ref)
    h = jnp.dot(x_ref[...], w1t_ref[...],
                preferred_element_type=jnp.float32) * dis
    _store_halves(h1_ref, h)


def _tc_mid_body(p_ref, h1_ref, hist_ref, w2t_ref, b1_ref, h2_ref):
    dis = _dis_block(hist_ref)
    o1 = dis * _cat_halves(p_ref, h1_ref) + b1_ref[...]
    h2 = jnp.dot(o1, w2t_ref[...], preferred_element_type=jnp.float32) * dis
    _store_halves(h2_ref, h2)


def _tc_final_body(p_ref, h2_ref, hist_ref, b2_ref, batch_ref, pool_ref):
    dis = _dis_block(hist_ref)
    o2 = dis * _cat_halves(p_ref, h2_ref) + b2_ref[...]
    gids = lax.broadcasted_iota(jnp.int32, (_RB, NUM_GRAPHS), 1)
    p1hot = (batch_ref[...] == gids).astype(jnp.float32)

    @pl.when(pl.program_id(0) == 0)
    def _():
        pool_ref[...] = jnp.zeros_like(pool_ref)

    pool_ref[...] += lax.dot_general(
        p1hot, o2, (((0,), (0,)), ((), ())),
        preferred_element_type=jnp.float32)


_x_spec = pl.BlockSpec((_RB, D), lambda i: (i, 0))
_half_spec = pl.BlockSpec((NC, _RB, D2), lambda i: (0, i, 0))
_hist_spec = pl.BlockSpec((NC, _RB, NS), lambda i: (0, i, 0))
_w_spec = pl.BlockSpec((D, D), lambda i: (0, 0))
_b_spec = pl.BlockSpec((1, D), lambda i: (0, 0))
_half_shape = jax.ShapeDtypeStruct((NC, N_NODES, D2), jnp.float32)


def kernel(x, edge_index, batch, W1, b1, W2, b2):
    edges = edge_index.astype(jnp.int32).reshape(2, NS, NCHUNK, K)
    batch2d = batch.astype(jnp.int32).reshape(N_NODES, 1)
    w1t = W1.T
    w2t = W2.T
    b1r = b1.reshape(1, D)
    b2r = b2.reshape(1, D)

    hist = _sc_degree(edges)

    h1 = pl.pallas_call(
        _tc_h1_body,
        grid=(_GRID,),
        in_specs=[_x_spec, _w_spec, _hist_spec],
        out_specs=_half_spec,
        out_shape=_half_shape,
    )(x, w1t, hist)

    s1 = _sc_edge_pass(h1, edges)

    h2 = pl.pallas_call(
        _tc_mid_body,
        grid=(_GRID,),
        in_specs=[_half_spec, _half_spec, _hist_spec, _w_spec, _b_spec],
        out_specs=_half_spec,
        out_shape=_half_shape,
    )(s1, h1, hist, w2t, b1r)

    s2 = _sc_edge_pass(h2, edges)

    pool = pl.pallas_call(
        _tc_final_body,
        grid=(_GRID,),
        in_specs=[_half_spec, _half_spec, _hist_spec, _b_spec,
                  pl.BlockSpec((_RB, 1), lambda i: (i, 0))],
        out_specs=pl.BlockSpec((NUM_GRAPHS, D), lambda i: (0, 0)),
        out_shape=jax.ShapeDtypeStruct((NUM_GRAPHS, D), jnp.float32),
    )(s2, h2, hist, b2r, batch2d)

    return pool


# speedup vs baseline: 39.9045x; 1.1970x over previous
import functools

import jax
import jax.numpy as jnp
from jax import lax
from jax.experimental import pallas as pl
from jax.experimental.pallas import tpu as pltpu
from jax.experimental.pallas import tpu_sc as plsc

N_NODES = 10000
NH = N_NODES // 2
N_EDGES = 320000
D = 128
D2 = D // 2
NUM_GRAPHS = 128

NC = 2
NS = 16
K = 80
NCHUNK = 250
WB = 624
TAIL = N_NODES - NS * WB
TAIL_START = NS * WB
ZROWS = 208

NBUF = 6
_NGRP = NCHUNK // NBUF
_NTAIL = NCHUNK - _NGRP * NBUF

_mesh = plsc.VectorSubcoreMesh(core_axis_name="c", subcore_axis_name="s")
_sc_params = pltpu.CompilerParams(use_tc_tiling_on_sc=False)


@functools.partial(
    pl.kernel,
    out_type=jax.ShapeDtypeStruct((NC, N_NODES, D2), jnp.float32),
    mesh=_mesh,
    scratch_types=[
        pltpu.VMEM((NCHUNK, K), jnp.int32),
        pltpu.VMEM((K, NS), jnp.float32),
        pltpu.VMEM((ZROWS, NS), jnp.float32),
        pltpu.VMEM((ZROWS, D2), jnp.float32),
        pltpu.VMEM_SHARED((N_NODES, NS), jnp.float32),
    ],
    compiler_params=_sc_params,
)
def _sc_degree(edge_hbm, out_hbm, cv, ones_v, hv, xb, hist):
    c = lax.axis_index("c")
    s = lax.axis_index("s")

    @pl.loop(0, K)
    def _(r):
        ones_v[r] = jnp.ones((NS,), jnp.float32)

    @pl.loop(0, ZROWS)
    def _(r):
        hv[r] = jnp.zeros((NS,), jnp.float32)

    @pl.loop(0, WB // ZROWS)
    def _(k):
        pltpu.sync_copy(hv, hist.at[pl.ds(s * WB + k * ZROWS, ZROWS)])

    @pl.when(s == 0)
    def _():
        pltpu.sync_copy(hv.at[pl.ds(0, TAIL)], hist.at[pl.ds(TAIL_START, TAIL)])

    pltpu.sync_copy(edge_hbm.at[1, s], cv)
    plsc.subcore_barrier()

    @pl.loop(0, NCHUNK // NC)
    def _(j):
        pltpu.sync_copy(ones_v, hist.at[cv.at[c * (NCHUNK // NC) + j]], add=True)

    plsc.subcore_barrier()

    @pl.loop(0, WB // ZROWS)
    def _(k):
        base = s * WB + k * ZROWS
        pltpu.sync_copy(hist.at[pl.ds(base, ZROWS)], hv)

        @pl.loop(0, ZROWS)
        def _(r):
            v = hv[r]

            @pl.loop(0, D2, step=NS)
            def _(cc):
                xb[r, pl.ds(cc, NS)] = v

        pltpu.sync_copy(xb, out_hbm.at[c, pl.ds(base, ZROWS)])

    @pl.when(s == 0)
    def _():
        pltpu.sync_copy(hist.at[pl.ds(TAIL_START, TAIL)], hv.at[pl.ds(0, TAIL)])

        @pl.loop(0, TAIL)
        def _(r):
            v = hv[r]

            @pl.loop(0, D2, step=NS)
            def _(cc):
                xb[r, pl.ds(cc, NS)] = v

        pltpu.sync_copy(xb.at[pl.ds(0, TAIL)],
                        out_hbm.at[c, pl.ds(TAIL_START, TAIL)])


@functools.partial(
    pl.kernel,
    out_type=jax.ShapeDtypeStruct((NC, N_NODES, D2), jnp.float32),
    mesh=_mesh,
    scratch_types=[
        pltpu.VMEM((NCHUNK, K), jnp.int32),
        pltpu.VMEM((NCHUNK, K), jnp.int32),
        pltpu.VMEM((NBUF, K, D2), jnp.float32),
        pltpu.VMEM((ZROWS, D2), jnp.float32),
        pltpu.VMEM_SHARED((N_NODES, D2), jnp.float32),
        pltpu.SemaphoreType.DMA((NBUF,)),
        pltpu.SemaphoreType.DMA((NBUF,)),
    ],
    compiler_params=_sc_params,
)
def _sc_edge_pass(h_hbm, edge_hbm, out_hbm,
                  rv, cv, bufs, zb, accum, gsem, ssem):
    c = lax.axis_index("c")
    s = lax.axis_index("s")
    hh = h_hbm.at[c]

    @pl.loop(0, ZROWS)
    def _(r):
        @pl.loop(0, D2, step=NS)
        def _(cc):
            zb[r, pl.ds(cc, NS)] = jnp.zeros((NS,), jnp.float32)

    @pl.loop(0, WB // ZROWS)
    def _(k):
        pltpu.sync_copy(zb, accum.at[pl.ds(s * WB + k * ZROWS, ZROWS)])

    @pl.when(s == 0)
    def _():
        pltpu.sync_copy(zb.at[pl.ds(0, TAIL)], accum.at[pl.ds(TAIL_START, TAIL)])

    pltpu.sync_copy(edge_hbm.at[0, s], rv)
    pltpu.sync_copy(edge_hbm.at[1, s], cv)
    plsc.subcore_barrier()

    for b in range(NBUF):
        pltpu.async_copy(hh.at[rv.at[b]], bufs.at[b], gsem.at[b])

    @pl.loop(0, _NGRP)
    def _(g):
        for b in range(NBUF):
            j = g * NBUF + b
            pltpu.make_async_copy(hh.at[rv.at[j]], bufs.at[b], gsem.at[b]).wait()
            pltpu.async_copy(bufs.at[b], accum.at[cv.at[j]], ssem.at[b], add=True)
            nxt = j + NBUF

            @pl.when(nxt < NCHUNK)
            def _():
                pltpu.make_async_copy(
                    bufs.at[b], accum.at[cv.at[j]], ssem.at[b]).wait()
                pltpu.async_copy(hh.at[rv.at[nxt]], bufs.at[b], gsem.at[b])

    for t in range(_NTAIL):
        j = _NGRP * NBUF + t
        pltpu.make_async_copy(hh.at[rv.at[j]], bufs.at[t], gsem.at[t]).wait()
        pltpu.async_copy(bufs.at[t], accum.at[cv.at[j]], ssem.at[t], add=True)

    for b in range(NBUF):
        pltpu.make_async_copy(bufs.at[b], accum.at[cv.at[0]], ssem.at[b]).wait()

    plsc.subcore_barrier()
    pltpu.sync_copy(accum.at[pl.ds(s * WB, WB)],
                    out_hbm.at[c, pl.ds(s * WB, WB)])

    @pl.when(s == 0)
    def _():
        pltpu.sync_copy(accum.at[pl.ds(TAIL_START, TAIL)],
                        out_hbm.at[c, pl.ds(TAIL_START, TAIL)])


_RB = 1000
_GRID = NH // _RB


def _dis_packed(hist_ref):
    return lax.rsqrt(hist_ref[0] + hist_ref[1] + 1.0)


def _pack(top, bot, out_ref):
    out_ref[0] = jnp.concatenate([top[:, :D2], bot[:, :D2]], axis=1)
    out_ref[1] = jnp.concatenate([top[:, D2:], bot[:, D2:]], axis=1)


def _unpack(p0, p1):
    top = jnp.concatenate([p0[:, :D2], p1[:, :D2]], axis=1)
    bot = jnp.concatenate([p0[:, D2:], p1[:, D2:]], axis=1)
    return top, bot


def _tc_h1_body(xt_ref, xb_ref, w1t_ref, hist_ref, h1_ref):
    dis = _dis_packed(hist_ref)
    w = w1t_ref[...]
    ht = jnp.dot(xt_ref[...], w, preferred_element_type=jnp.float32)
    hb = jnp.dot(xb_ref[...], w, preferred_element_type=jnp.float32)
    _pack(ht, hb, h1_ref)
    h1_ref[0] = h1_ref[0] * dis
    h1_ref[1] = h1_ref[1] * dis


def _tc_mid_body(p_ref, h1_ref, hist_ref, w2t_ref, b1_ref, h2_ref):
    dis = _dis_packed(hist_ref)
    o1p0 = dis * (p_ref[0] + h1_ref[0]) + b1_ref[0]
    o1p1 = dis * (p_ref[1] + h1_ref[1]) + b1_ref[1]
    o1t, o1b = _unpack(o1p0, o1p1)
    w = w2t_ref[...]
    ht = jnp.dot(o1t, w, preferred_element_type=jnp.float32)
    hb = jnp.dot(o1b, w, preferred_element_type=jnp.float32)
    _pack(ht, hb, h2_ref)
    h2_ref[0] = h2_ref[0] * dis
    h2_ref[1] = h2_ref[1] * dis


def _tc_final_body(p_ref, h2_ref, hist_ref, b2_ref, bt_ref, bb_ref, pool_ref):
    dis = _dis_packed(hist_ref)
    o2p0 = dis * (p_ref[0] + h2_ref[0]) + b2_ref[0]
    o2p1 = dis * (p_ref[1] + h2_ref[1]) + b2_ref[1]
    o2t, o2b = _unpack(o2p0, o2p1)
    gids = lax.broadcasted_iota(jnp.int32, (_RB, NUM_GRAPHS), 1)
    pt = (bt_ref[...] == gids).astype(jnp.float32)
    pb = (bb_ref[...] == gids).astype(jnp.float32)

    @pl.when(pl.program_id(0) == 0)
    def _():
        pool_ref[...] = jnp.zeros_like(pool_ref)

    dn = (((0,), (0,)), ((), ()))
    pool_ref[...] += (
        lax.dot_general(pt, o2t, dn, preferred_element_type=jnp.float32)
        + lax.dot_general(pb, o2b, dn, preferred_element_type=jnp.float32))


_xt_spec = pl.BlockSpec((_RB, D), lambda i: (i, 0))
_xb_spec = pl.BlockSpec((_RB, D), lambda i: (i + _GRID, 0))
_pk_spec = pl.BlockSpec((NC, _RB, D), lambda i: (0, i, 0))
_w_spec = pl.BlockSpec((D, D), lambda i: (0, 0))
_bp_spec = pl.BlockSpec((NC, 1, D), lambda i: (0, 0, 0))
_pk_shape = jax.ShapeDtypeStruct((NC, NH, D), jnp.float32)


def _packed_bias(b):
    bp = b.reshape(2, D2)
    return jnp.concatenate([bp, bp], axis=1).reshape(NC, 1, D)


def kernel(x, edge_index, batch, W1, b1, W2, b2):
    e = edge_index.astype(jnp.int32)
    eg = jnp.where(e < NH, e * 2, e * 2 - (N_NODES - 1))
    edges = eg.reshape(2, NS, NCHUNK, K)
    batch2d = batch.astype(jnp.int32).reshape(N_NODES, 1)
    w1t = W1.T
    w2t = W2.T
    b1p = _packed_bias(b1)
    b2p = _packed_bias(b2)

    histf = _sc_degree(edges)
    hist = histf.reshape(NC, NH, D)

    h1 = pl.pallas_call(
        _tc_h1_body,
        grid=(_GRID,),
        in_specs=[_xt_spec, _xb_spec, _w_spec, _pk_spec],
        out_specs=_pk_spec,
        out_shape=_pk_shape,
    )(x, x, w1t, hist)

    s1 = _sc_edge_pass(h1.reshape(NC, N_NODES, D2), edges)

    h2 = pl.pallas_call(
        _tc_mid_body,
        grid=(_GRID,),
        in_specs=[_pk_spec, _pk_spec, _pk_spec, _w_spec, _bp_spec],
        out_specs=_pk_spec,
        out_shape=_pk_shape,
    )(s1.reshape(NC, NH, D), h1, hist, w2t, b1p)

    s2 = _sc_edge_pass(h2.reshape(NC, N_NODES, D2), edges)

    pool = pl.pallas_call(
        _tc_final_body,
        grid=(_GRID,),
        in_specs=[_pk_spec, _pk_spec, _pk_spec, _bp_spec,
                  pl.BlockSpec((_RB, 1), lambda i: (i, 0)),
                  pl.BlockSpec((_RB, 1), lambda i: (i + _GRID, 0))],
        out_specs=pl.BlockSpec((NUM_GRAPHS, D), lambda i: (0, 0)),
        out_shape=jax.ShapeDtypeStruct((NUM_GRAPHS, D), jnp.float32),
    )(s2.reshape(NC, NH, D), h2, hist, b2p, batch2d, batch2d)

    return pool


# speedup vs baseline: 41.0193x; 1.0279x over previous
import functools

import jax
import jax.numpy as jnp
from jax import lax
from jax.experimental import pallas as pl
from jax.experimental.pallas import tpu as pltpu
from jax.experimental.pallas import tpu_sc as plsc

N_NODES = 10000
NH = N_NODES // 2
N_EDGES = 320000
D = 128
D2 = D // 2
NUM_GRAPHS = 128

NC = 2
NS = 16
K = 80
NCHUNK = 250
WB = 624
TAIL = N_NODES - NS * WB
TAIL_START = NS * WB
ZROWS = 208

NBUF = 6
_NGRP = NCHUNK // NBUF
_NTAIL = NCHUNK - _NGRP * NBUF

_mesh = plsc.VectorSubcoreMesh(core_axis_name="c", subcore_axis_name="s")
_sc_params = pltpu.CompilerParams(use_tc_tiling_on_sc=False)


@functools.partial(
    pl.kernel,
    out_type=jax.ShapeDtypeStruct((NC, N_NODES, D2), jnp.float32),
    mesh=_mesh,
    scratch_types=[
        pltpu.VMEM((NCHUNK, K), jnp.int32),
        pltpu.VMEM((K, NS), jnp.float32),
        pltpu.VMEM((ZROWS, NS), jnp.float32),
        pltpu.VMEM((ZROWS, D2), jnp.float32),
        pltpu.VMEM_SHARED((N_NODES, NS), jnp.float32),
        pltpu.SemaphoreType.DMA,
    ],
    compiler_params=_sc_params,
)
def _sc_degree(edge_hbm, out_hbm, cv, ones_v, hv, xb, hist, ssem):
    c = lax.axis_index("c")
    s = lax.axis_index("s")

    @pl.loop(0, K)
    def _(r):
        ones_v[r] = jnp.ones((NS,), jnp.float32)

    @pl.loop(0, ZROWS)
    def _(r):
        hv[r] = jnp.zeros((NS,), jnp.float32)

    @pl.loop(0, WB // ZROWS)
    def _(k):
        pltpu.sync_copy(hv, hist.at[pl.ds(s * WB + k * ZROWS, ZROWS)])

    @pl.when(s == 0)
    def _():
        pltpu.sync_copy(hv.at[pl.ds(0, TAIL)], hist.at[pl.ds(TAIL_START, TAIL)])

    pltpu.sync_copy(edge_hbm.at[1, s], cv)
    plsc.subcore_barrier()

    @pl.loop(0, NCHUNK // NC)
    def _(j):
        pltpu.async_copy(ones_v, hist.at[cv.at[c * (NCHUNK // NC) + j]],
                         ssem, add=True)

    @pl.loop(0, NCHUNK // NC)
    def _(j):
        pltpu.make_async_copy(ones_v, hist.at[cv.at[j]], ssem).wait()

    plsc.subcore_barrier()

    @pl.loop(0, WB // ZROWS)
    def _(k):
        base = s * WB + k * ZROWS
        pltpu.sync_copy(hist.at[pl.ds(base, ZROWS)], hv)

        @pl.loop(0, ZROWS)
        def _(r):
            v = hv[r]

            @pl.loop(0, D2, step=NS)
            def _(cc):
                xb[r, pl.ds(cc, NS)] = v

        pltpu.sync_copy(xb, out_hbm.at[c, pl.ds(base, ZROWS)])

    @pl.when(s == 0)
    def _():
        pltpu.sync_copy(hist.at[pl.ds(TAIL_START, TAIL)], hv.at[pl.ds(0, TAIL)])

        @pl.loop(0, TAIL)
        def _(r):
            v = hv[r]

            @pl.loop(0, D2, step=NS)
            def _(cc):
                xb[r, pl.ds(cc, NS)] = v

        pltpu.sync_copy(xb.at[pl.ds(0, TAIL)],
                        out_hbm.at[c, pl.ds(TAIL_START, TAIL)])


@functools.partial(
    pl.kernel,
    out_type=jax.ShapeDtypeStruct((NC, N_NODES, D2), jnp.float32),
    mesh=_mesh,
    scratch_types=[
        pltpu.VMEM((NCHUNK, K), jnp.int32),
        pltpu.VMEM((NCHUNK, K), jnp.int32),
        pltpu.VMEM((NBUF, K, D2), jnp.float32),
        pltpu.VMEM((ZROWS, D2), jnp.float32),
        pltpu.VMEM_SHARED((N_NODES, D2), jnp.float32),
        pltpu.SemaphoreType.DMA((NBUF,)),
        pltpu.SemaphoreType.DMA((NBUF,)),
    ],
    compiler_params=_sc_params,
)
def _sc_edge_pass(h_hbm, edge_hbm, out_hbm,
                  rv, cv, bufs, zb, accum, gsem, ssem):
    c = lax.axis_index("c")
    s = lax.axis_index("s")
    hh = h_hbm.at[c]

    @pl.loop(0, ZROWS)
    def _(r):
        @pl.loop(0, D2, step=NS)
        def _(cc):
            zb[r, pl.ds(cc, NS)] = jnp.zeros((NS,), jnp.float32)

    @pl.loop(0, WB // ZROWS)
    def _(k):
        pltpu.sync_copy(zb, accum.at[pl.ds(s * WB + k * ZROWS, ZROWS)])

    @pl.when(s == 0)
    def _():
        pltpu.sync_copy(zb.at[pl.ds(0, TAIL)], accum.at[pl.ds(TAIL_START, TAIL)])

    pltpu.sync_copy(edge_hbm.at[0, s], rv)
    pltpu.sync_copy(edge_hbm.at[1, s], cv)
    plsc.subcore_barrier()

    for b in range(NBUF):
        pltpu.async_copy(hh.at[rv.at[b]], bufs.at[b], gsem.at[b])

    @pl.loop(0, _NGRP)
    def _(g):
        for b in range(NBUF):
            j = g * NBUF + b
            pltpu.make_async_copy(hh.at[rv.at[j]], bufs.at[b], gsem.at[b]).wait()
            pltpu.async_copy(bufs.at[b], accum.at[cv.at[j]], ssem.at[b], add=True)
            nxt = j + NBUF

            @pl.when(nxt < NCHUNK)
            def _():
                pltpu.make_async_copy(
                    bufs.at[b], accum.at[cv.at[j]], ssem.at[b]).wait()
                pltpu.async_copy(hh.at[rv.at[nxt]], bufs.at[b], gsem.at[b])

    for t in range(_NTAIL):
        j = _NGRP * NBUF + t
        pltpu.make_async_copy(hh.at[rv.at[j]], bufs.at[t], gsem.at[t]).wait()
        pltpu.async_copy(bufs.at[t], accum.at[cv.at[j]], ssem.at[t], add=True)

    for b in range(NBUF):
        pltpu.make_async_copy(bufs.at[b], accum.at[cv.at[0]], ssem.at[b]).wait()

    plsc.subcore_barrier()
    pltpu.sync_copy(accum.at[pl.ds(s * WB, WB)],
                    out_hbm.at[c, pl.ds(s * WB, WB)])

    @pl.when(s == 0)
    def _():
        pltpu.sync_copy(accum.at[pl.ds(TAIL_START, TAIL)],
                        out_hbm.at[c, pl.ds(TAIL_START, TAIL)])


_RB = 1000
_GRID = NH // _RB


def _dis_packed(hist_ref):
    return lax.rsqrt(hist_ref[0] + hist_ref[1] + 1.0)


def _pack(top, bot, out_ref):
    out_ref[0] = jnp.concatenate([top[:, :D2], bot[:, :D2]], axis=1)
    out_ref[1] = jnp.concatenate([top[:, D2:], bot[:, D2:]], axis=1)


def _unpack(p0, p1):
    top = jnp.concatenate([p0[:, :D2], p1[:, :D2]], axis=1)
    bot = jnp.concatenate([p0[:, D2:], p1[:, D2:]], axis=1)
    return top, bot


def _tc_h1_body(xt_ref, xb_ref, w1t_ref, hist_ref, h1_ref):
    dis = _dis_packed(hist_ref)
    w = w1t_ref[...]
    ht = jnp.dot(xt_ref[...], w, preferred_element_type=jnp.float32)
    hb = jnp.dot(xb_ref[...], w, preferred_element_type=jnp.float32)
    _pack(ht, hb, h1_ref)
    h1_ref[0] = h1_ref[0] * dis
    h1_ref[1] = h1_ref[1] * dis


def _tc_mid_body(p_ref, h1_ref, hist_ref, w2t_ref, b1_ref, h2_ref):
    dis = _dis_packed(hist_ref)
    o1p0 = dis * (p_ref[0] + h1_ref[0]) + b1_ref[0]
    o1p1 = dis * (p_ref[1] + h1_ref[1]) + b1_ref[1]
    o1t, o1b = _unpack(o1p0, o1p1)
    w = w2t_ref[...]
    ht = jnp.dot(o1t, w, preferred_element_type=jnp.float32)
    hb = jnp.dot(o1b, w, preferred_element_type=jnp.float32)
    _pack(ht, hb, h2_ref)
    h2_ref[0] = h2_ref[0] * dis
    h2_ref[1] = h2_ref[1] * dis


def _tc_final_body(p_ref, h2_ref, hist_ref, b2_ref, bt_ref, bb_ref, pool_ref):
    dis = _dis_packed(hist_ref)
    o2p0 = dis * (p_ref[0] + h2_ref[0]) + b2_ref[0]
    o2p1 = dis * (p_ref[1] + h2_ref[1]) + b2_ref[1]
    o2t, o2b = _unpack(o2p0, o2p1)
    gids = lax.broadcasted_iota(jnp.int32, (_RB, NUM_GRAPHS), 1)
    pt = (bt_ref[...] == gids).astype(jnp.float32)
    pb = (bb_ref[...] == gids).astype(jnp.float32)

    @pl.when(pl.program_id(0) == 0)
    def _():
        pool_ref[...] = jnp.zeros_like(pool_ref)

    dn = (((0,), (0,)), ((), ()))
    pool_ref[...] += (
        lax.dot_general(pt, o2t, dn, preferred_element_type=jnp.float32)
        + lax.dot_general(pb, o2b, dn, preferred_element_type=jnp.float32))


_xt_spec = pl.BlockSpec((_RB, D), lambda i: (i, 0))
_xb_spec = pl.BlockSpec((_RB, D), lambda i: (i + _GRID, 0))
_pk_spec = pl.BlockSpec((NC, _RB, D), lambda i: (0, i, 0))
_w_spec = pl.BlockSpec((D, D), lambda i: (0, 0))
_bp_spec = pl.BlockSpec((NC, 1, D), lambda i: (0, 0, 0))
_pk_shape = jax.ShapeDtypeStruct((NC, NH, D), jnp.float32)


def _packed_bias(b):
    bp = b.reshape(2, D2)
    return jnp.concatenate([bp, bp], axis=1).reshape(NC, 1, D)


def kernel(x, edge_index, batch, W1, b1, W2, b2):
    e = edge_index.astype(jnp.int32)
    eg = jnp.where(e < NH, e * 2, e * 2 - (N_NODES - 1))
    edges = eg.reshape(2, NS, NCHUNK, K)
    batch2d = batch.astype(jnp.int32).reshape(N_NODES, 1)
    w1t = W1.T
    w2t = W2.T
    b1p = _packed_bias(b1)
    b2p = _packed_bias(b2)

    histf = _sc_degree(edges)
    hist = histf.reshape(NC, NH, D)

    h1 = pl.pallas_call(
        _tc_h1_body,
        grid=(_GRID,),
        in_specs=[_xt_spec, _xb_spec, _w_spec, _pk_spec],
        out_specs=_pk_spec,
        out_shape=_pk_shape,
    )(x, x, w1t, hist)

    s1 = _sc_edge_pass(h1.reshape(NC, N_NODES, D2), edges)

    h2 = pl.pallas_call(
        _tc_mid_body,
        grid=(_GRID,),
        in_specs=[_pk_spec, _pk_spec, _pk_spec, _w_spec, _bp_spec],
        out_specs=_pk_spec,
        out_shape=_pk_shape,
    )(s1.reshape(NC, NH, D), h1, hist, w2t, b1p)

    s2 = _sc_edge_pass(h2.reshape(NC, N_NODES, D2), edges)

    pool = pl.pallas_call(
        _tc_final_body,
        grid=(_GRID,),
        in_specs=[_pk_spec, _pk_spec, _pk_spec, _bp_spec,
                  pl.BlockSpec((_RB, 1), lambda i: (i, 0)),
                  pl.BlockSpec((_RB, 1), lambda i: (i + _GRID, 0))],
        out_specs=pl.BlockSpec((NUM_GRAPHS, D), lambda i: (0, 0)),
        out_shape=jax.ShapeDtypeStruct((NUM_GRAPHS, D), jnp.float32),
    )(s2.reshape(NC, NH, D), h2, hist, b2p, batch2d, batch2d)

    return pool
